# baseline (speedup 1.0000x reference)
import functools

import jax
import jax.numpy as jnp
from jax import lax
from jax.experimental import pallas as pl
from jax.experimental.pallas import tpu as pltpu
from jax.experimental.pallas import tpu_sc as plsc

D = 128
H = 150
CTX = 200
L = 16
RPT = 8
NSUB = 16
NCHUNK = CTX // RPT
NB = NCHUNK - NSUB
NH = 10
ND = D // L
_COL0 = H - L

_LN2 = 0.6931471805599453
_SQRTH = 0.70710678118654752440


def _ln_vec(x):
    m = x
    e = jnp.zeros((L,), jnp.float32)
    one = jnp.float32(1.0)
    half = jnp.float32(0.5)
    for _ in range(8):
        big = m >= jnp.float32(2.0)
        m = jnp.where(big, m * half, m)
        e = jnp.where(big, e + one, e)
    big = m > jnp.float32(2.0 * _SQRTH)
    m = jnp.where(big, m * half, m)
    e = jnp.where(big, e + one, e)
    f = m - one
    z = f * f
    p = jnp.full((L,), 7.0376836292e-2, jnp.float32)
    for c in (-1.1514610310e-1, 1.1676998740e-1, -1.2420140846e-1,
              1.4249322787e-1, -1.6668057665e-1, 2.0000714765e-1,
              -2.4999993993e-1, 3.3333331174e-1):
        p = p * f + jnp.float32(c)
    y = f * z * p - half * z + f
    return y + e * jnp.float32(_LN2)


def _lane_reduce(x, op):
    lane = lax.iota(jnp.int32, L)
    dnums = lax.GatherDimensionNumbers(
        offset_dims=(), collapsed_slice_dims=(0,), start_index_map=(0,))
    for sh in (8, 4, 2, 1):
        perm = (lane + sh) & (L - 1)
        shuf = lax.gather(x, perm[:, None], dnums, slice_sizes=(1,),
                          mode=lax.GatherScatterMode.PROMISE_IN_BOUNDS)
        x = op(x, shuf)
    return x


def _sc_body(idx_hbm, table_hbm, w1_hbm, b1_hbm, w2_hbm, b2_hbm, out_hbm,
             idxa_v, idxb_v, rowsa_v, rowsb_v, part_v, allp_v, pooled_v,
             w1c_v, w1f_v, b1f_v, w2r_v, w2f_v, b2f_v, plog_v, out_v,
             part_sh, plog_sh, sem_i, sem_g, sem_w1, sem_w2, sem_b2):
    s = lax.axis_index("s")
    col = pl.multiple_of(s * L, L)

    @pl.when(s < NH - 1)
    def _pf_main():
        pltpu.make_async_copy(w1_hbm.at[:, pl.ds(col, L)], w1c_v,
                              sem_w1).start()
        pltpu.make_async_copy(b1_hbm, b1f_v, sem_w1).start()
        pltpu.make_async_copy(w2_hbm.at[pl.ds(col, L), :], w2r_v,
                              sem_w2).start()

    @pl.when(s == NH - 1)
    def _pf_tail():
        pltpu.make_async_copy(w1_hbm, w1f_v, sem_w1).start()
        pltpu.make_async_copy(b1_hbm, b1f_v, sem_w1).start()
        pltpu.make_async_copy(w2_hbm, w2f_v, sem_w2).start()

    @pl.when(s == 0)
    def _pf_b2():
        pltpu.make_async_copy(b2_hbm, b2f_v, sem_b2).start()

    pltpu.make_async_copy(idx_hbm.at[pl.ds(s * RPT, RPT)], idxa_v,
                          sem_i).start()

    @pl.when(s < NB)
    def _idx_b():
        pltpu.make_async_copy(idx_hbm.at[pl.ds((NSUB + s) * RPT, RPT)],
                              idxb_v, sem_i).start()

    pltpu.make_async_copy(idx_hbm.at[pl.ds(s * RPT, RPT)], idxa_v,
                          sem_i).wait()
    pltpu.make_async_copy(table_hbm.at[idxa_v], rowsa_v, sem_g).start()

    @pl.when(s < NB)
    def _gather_b():
        pltpu.make_async_copy(idx_hbm.at[pl.ds((NSUB + s) * RPT, RPT)],
                              idxb_v, sem_i).wait()
        pltpu.make_async_copy(table_hbm.at[idxb_v], rowsb_v, sem_g).start()

    pltpu.make_async_copy(table_hbm.at[idxa_v], rowsa_v, sem_g).wait()

    @pl.when(s >= NB)
    def _pool_a():
        for k in range(ND):
            acc = rowsa_v[0, pl.ds(k * L, L)]
            for r in range(1, RPT):
                acc = acc + rowsa_v[r, pl.ds(k * L, L)]
            part_v[pl.ds(k * L, L)] = acc

    @pl.when(s < NB)
    def _pool_ab():
        pltpu.make_async_copy(table_hbm.at[idxb_v], rowsb_v, sem_g).wait()
        for k in range(ND):
            acc = rowsa_v[0, pl.ds(k * L, L)]
            for r in range(1, RPT):
                acc = acc + rowsa_v[r, pl.ds(k * L, L)]
            for r in range(RPT):
                acc = acc + rowsb_v[r, pl.ds(k * L, L)]
            part_v[pl.ds(k * L, L)] = acc

    pltpu.sync_copy(part_v, part_sh.at[s])
    plsc.subcore_barrier()

    def _hidden_and_plog(w1win, w2rows, lanes, j0):
        pltpu.sync_copy(part_sh, allp_v)
        for k in range(ND):
            acc = allp_v[0, pl.ds(k * L, L)]
            for r in range(1, NSUB):
                acc = acc + allp_v[r, pl.ds(k * L, L)]
            pooled_v[pl.ds(k * L, L)] = acc
        hacc = b1f_v[pl.ds(j0, L)] if j0 is not None else b1f_v[pl.ds(col, L)]
        for g in range(ND):
            pv = pooled_v[pl.ds(g * L, L)]
            for i in range(L):
                hacc = hacc + pv[i] * w1win(g * L + i)
        h = jnp.maximum(hacc, 0.0)
        for k in range(ND):
            acc = jnp.zeros((L,), jnp.float32)
            for lane in lanes:
                acc = acc + h[lane] * w2rows(lane, k)
            part_v[pl.ds(k * L, L)] = acc

    @pl.when(s < NH - 1)
    def _mid_main():
        pltpu.make_async_copy(w1_hbm.at[:, pl.ds(col, L)], w1c_v,
                              sem_w1).wait()
        pltpu.make_async_copy(b1_hbm, b1f_v, sem_w1).wait()
        pltpu.make_async_copy(w2_hbm.at[pl.ds(col, L), :], w2r_v,
                              sem_w2).wait()
        _hidden_and_plog(
            lambda row: w1c_v[row, :],
            lambda lane, k: w2r_v[lane, pl.ds(k * L, L)],
            range(L), None)
        pltpu.sync_copy(part_v, plog_sh.at[s])

    @pl.when(s == NH - 1)
    def _mid_tail():
        pltpu.make_async_copy(w1_hbm, w1f_v, sem_w1).wait()
        pltpu.make_async_copy(b1_hbm, b1f_v, sem_w1).wait()
        pltpu.make_async_copy(w2_hbm, w2f_v, sem_w2).wait()
        _hidden_and_plog(
            lambda row: w1f_v[row, pl.ds(_COL0, L)],
            lambda lane, k: w2f_v[_COL0 + lane, pl.ds(k * L, L)],
            range((NH - 1) * L - _COL0, H - _COL0), _COL0)
        pltpu.sync_copy(part_v, plog_sh.at[NH - 1])

    plsc.subcore_barrier()

    @pl.when(s == 0)
    def _softmax():
        pltpu.sync_copy(plog_sh, plog_v)
        pltpu.make_async_copy(b2_hbm, b2f_v, sem_b2).wait()
        lgs = []
        for k in range(ND):
            acc = b2f_v[pl.ds(k * L, L)]
            for t in range(NH):
                acc = acc + plog_v[t, pl.ds(k * L, L)]
            lgs.append(acc)
        mv = lgs[0]
        for k in range(1, ND):
            mv = jnp.maximum(mv, lgs[k])
        m = _lane_reduce(mv, jnp.maximum)
        tot = jnp.zeros((L,), jnp.float32)
        for k in range(ND):
            tot = tot + jnp.exp(lgs[k] - m)
        lse = _ln_vec(_lane_reduce(tot, jnp.add)) + m
        for k in range(ND):
            out_v[pl.ds(k * L, L)] = lgs[k] - lse
        pltpu.sync_copy(out_v, out_hbm.at[0])


@functools.cache
def _sc_cbow():
    return pl.kernel(
        _sc_body,
        mesh=plsc.VectorSubcoreMesh(core_axis_name="c", subcore_axis_name="s",
                                    num_cores=1),
        compiler_params=pltpu.CompilerParams(use_tc_tiling_on_sc=False),
        out_type=jax.ShapeDtypeStruct((1, D), jnp.float32),
        scratch_types=[
            pltpu.VMEM((RPT,), jnp.int32),
            pltpu.VMEM((RPT,), jnp.int32),
            pltpu.VMEM((RPT, D), jnp.float32),
            pltpu.VMEM((RPT, D), jnp.float32),
            pltpu.VMEM((D,), jnp.float32),
            pltpu.VMEM((NSUB, D), jnp.float32),
            pltpu.VMEM((D,), jnp.float32),
            pltpu.VMEM((D, L), jnp.float32),
            pltpu.VMEM((D, H), jnp.float32),
            pltpu.VMEM((H,), jnp.float32),
            pltpu.VMEM((L, D), jnp.float32),
            pltpu.VMEM((H, D), jnp.float32),
            pltpu.VMEM((D,), jnp.float32),
            pltpu.VMEM((NH, D), jnp.float32),
            pltpu.VMEM((D,), jnp.float32),
            pltpu.VMEM_SHARED((NSUB, D), jnp.float32),
            pltpu.VMEM_SHARED((NH, D), jnp.float32),
            pltpu.SemaphoreType.DMA,
            pltpu.SemaphoreType.DMA,
            pltpu.SemaphoreType.DMA,
            pltpu.SemaphoreType.DMA,
            pltpu.SemaphoreType.DMA,
        ],
    )


def kernel(input, emb_table, W1, b1, W2, b2):
    idx = input.astype(jnp.int32)
    return _sc_cbow()(idx, emb_table, W1, b1, W2, b2)


# speedup vs baseline: 1.1099x; 1.1099x over previous
import functools

import jax
import jax.numpy as jnp
from jax import lax
from jax.experimental import pallas as pl
from jax.experimental.pallas import tpu as pltpu
from jax.experimental.pallas import tpu_sc as plsc

D = 128
H = 150
CTX = 200
L = 16
RPT = 8
NSUB = 16
NCHUNK = CTX // RPT
NB = NCHUNK - NSUB
ND = D // L


def _sc_body(idx_hbm, table_hbm, out_hbm,
             idxa_v, idxb_v, rowsa_v, rowsb_v, part_v, sem_i, sem_g):
    s = lax.axis_index("s")

    pltpu.make_async_copy(idx_hbm.at[pl.ds(s * RPT, RPT)], idxa_v,
                          sem_i).start()

    @pl.when(s < NB)
    def _idx_b():
        pltpu.make_async_copy(idx_hbm.at[pl.ds((NSUB + s) * RPT, RPT)],
                              idxb_v, sem_i).start()

    pltpu.make_async_copy(idx_hbm.at[pl.ds(s * RPT, RPT)], idxa_v,
                          sem_i).wait()
    pltpu.make_async_copy(table_hbm.at[idxa_v], rowsa_v, sem_g).start()

    @pl.when(s < NB)
    def _gather_b():
        pltpu.make_async_copy(idx_hbm.at[pl.ds((NSUB + s) * RPT, RPT)],
                              idxb_v, sem_i).wait()
        pltpu.make_async_copy(table_hbm.at[idxb_v], rowsb_v, sem_g).start()

    pltpu.make_async_copy(table_hbm.at[idxa_v], rowsa_v, sem_g).wait()

    @pl.when(s >= NB)
    def _pool_a():
        for k in range(ND):
            acc = rowsa_v[0, pl.ds(k * L, L)]
            for r in range(1, RPT):
                acc = acc + rowsa_v[r, pl.ds(k * L, L)]
            part_v[pl.ds(k * L, L)] = acc

    @pl.when(s < NB)
    def _pool_ab():
        pltpu.make_async_copy(table_hbm.at[idxb_v], rowsb_v, sem_g).wait()
        for k in range(ND):
            acc = rowsa_v[0, pl.ds(k * L, L)]
            for r in range(1, RPT):
                acc = acc + rowsa_v[r, pl.ds(k * L, L)]
            for r in range(RPT):
                acc = acc + rowsb_v[r, pl.ds(k * L, L)]
            part_v[pl.ds(k * L, L)] = acc

    pltpu.sync_copy(part_v, out_hbm.at[s])


@functools.cache
def _sc_pool():
    return pl.kernel(
        _sc_body,
        mesh=plsc.VectorSubcoreMesh(core_axis_name="c", subcore_axis_name="s",
                                    num_cores=1),
        compiler_params=pltpu.CompilerParams(use_tc_tiling_on_sc=False),
        out_type=jax.ShapeDtypeStruct((NSUB, D), jnp.float32),
        scratch_types=[
            pltpu.VMEM((RPT,), jnp.int32),
            pltpu.VMEM((RPT,), jnp.int32),
            pltpu.VMEM((RPT, D), jnp.float32),
            pltpu.VMEM((RPT, D), jnp.float32),
            pltpu.VMEM((D,), jnp.float32),
            pltpu.SemaphoreType.DMA,
            pltpu.SemaphoreType.DMA,
        ],
    )


def _mlp_body(p_ref, w1_ref, b1_ref, w2_ref, b2_ref, out_ref):
    pooled = jnp.sum(p_ref[...], axis=0, keepdims=True)
    h = jnp.dot(pooled, w1_ref[...], preferred_element_type=jnp.float32)
    h = jnp.maximum(h + b1_ref[...], 0.0)
    logits = jnp.dot(h, w2_ref[...], preferred_element_type=jnp.float32)
    logits = logits + b2_ref[...]
    m = jnp.max(logits, axis=-1, keepdims=True)
    x = logits - m
    lse = jnp.log(jnp.sum(jnp.exp(x), axis=-1, keepdims=True))
    out_ref[...] = x - lse


_mlp = pl.pallas_call(
    _mlp_body,
    out_shape=jax.ShapeDtypeStruct((1, D), jnp.float32),
)


def kernel(input, emb_table, W1, b1, W2, b2):
    idx = input.astype(jnp.int32)
    parts = _sc_pool()(idx, emb_table)
    return _mlp(parts, W1, b1.reshape(1, H), W2, b2.reshape(1, D))
